# Initial kernel scaffold; baseline (speedup 1.0000x reference)
import jax
import jax.numpy as jnp
from jax.experimental import pallas as pl


def kernel(x):
    raise NotImplementedError("write your pallas kernel here")


# speedup vs baseline: 21.3249x; 21.3249x over previous
import functools

import jax
import jax.numpy as jnp
from jax.experimental import pallas as pl
from jax.experimental.pallas import tpu as pltpu


def _pool_body(x_ref, o_ref, *, inv_hw, c):
    xb = x_ref[...]
    s = jnp.sum(xb, axis=0)
    m = jnp.max(xb, axis=0)
    o_ref[:, :c] = s * inv_hw
    o_ref[:, c:] = m


def kernel(x):
    n, c, h, w = x.shape
    hw = h * w
    dtype = x.dtype

    xt = x.transpose(2, 3, 0, 1).reshape(hw, n, c)

    bn = 16 if n % 16 == 0 else 8
    grid = (n // bn,)

    in_block_bytes = hw * bn * c * jnp.dtype(dtype).itemsize
    vmem_limit = int(min(2 * in_block_bytes + (2 << 20) + (16 << 20), 100 << 20))

    out = pl.pallas_call(
        functools.partial(_pool_body, inv_hw=1.0 / hw, c=c),
        out_shape=jax.ShapeDtypeStruct((n, 2 * c), dtype),
        grid=grid,
        in_specs=[pl.BlockSpec((hw, bn, c), lambda i: (0, i, 0))],
        out_specs=pl.BlockSpec((bn, 2 * c), lambda i: (i, 0)),
        compiler_params=pltpu.CompilerParams(
            dimension_semantics=("parallel",),
            vmem_limit_bytes=vmem_limit,
        ),
        cost_estimate=pl.CostEstimate(
            flops=2 * n * c * hw,
            transcendentals=0,
            bytes_accessed=n * c * hw * jnp.dtype(dtype).itemsize,
        ),
    )(xt)

    return out.reshape(n, 2 * c, 1, 1)


# speedup vs baseline: 23.2311x; 1.0894x over previous
import functools

import jax
import jax.numpy as jnp
from jax.experimental import pallas as pl
from jax.experimental.pallas import tpu as pltpu


def _pool_body(x_ref, o_ref, *, inv_hw, c):
    xb = x_ref[...]
    s = jnp.sum(xb, axis=0)
    m = jnp.max(xb, axis=0)
    o_ref[:, :c] = s * inv_hw
    o_ref[:, c:] = m


def kernel(x):
    n, c, h, w = x.shape
    hw = h * w
    dtype = x.dtype

    xt = x.transpose(2, 3, 0, 1).reshape(hw, n, c)

    bn = 32 if n % 32 == 0 else (16 if n % 16 == 0 else 8)
    grid = (n // bn,)

    in_block_bytes = hw * bn * c * jnp.dtype(dtype).itemsize
    vmem_limit = int(min(2 * in_block_bytes + (2 << 20) + (16 << 20), 100 << 20))

    out = pl.pallas_call(
        functools.partial(_pool_body, inv_hw=1.0 / hw, c=c),
        out_shape=jax.ShapeDtypeStruct((n, 2 * c), dtype),
        grid=grid,
        in_specs=[pl.BlockSpec((hw, bn, c), lambda i: (0, i, 0))],
        out_specs=pl.BlockSpec((bn, 2 * c), lambda i: (i, 0)),
        compiler_params=pltpu.CompilerParams(
            dimension_semantics=("parallel",),
            vmem_limit_bytes=vmem_limit,
        ),
        cost_estimate=pl.CostEstimate(
            flops=2 * n * c * hw,
            transcendentals=0,
            bytes_accessed=n * c * hw * jnp.dtype(dtype).itemsize,
        ),
    )(xt)

    return out.reshape(n, 2 * c, 1, 1)
